# Initial kernel scaffold; baseline (speedup 1.0000x reference)
import jax
import jax.numpy as jnp
from jax.experimental import pallas as pl


def kernel(x, W_lin, b_lin, W_emb):
    raise NotImplementedError("write your pallas kernel here")


# speedup vs baseline: 33.5599x; 33.5599x over previous
import functools

import jax
import jax.numpy as jnp
import numpy as np
from jax import lax
from jax.experimental import pallas as pl
from jax.experimental.pallas import tpu as pltpu
from jax.experimental.pallas import tpu_sc as plsc

_FIELD_DIMS = [1000] * 26
_F = len(_FIELD_DIMS)
_V = sum(_FIELD_DIMS)
_D = 32
_B = 4096
_OFFS = np.array((0, *np.cumsum(_FIELD_DIMS)[:-1]), dtype=np.int32)
_I, _J = np.triu_indices(_F, k=1)
_NPAIR = _I.size
_NROW = 2 * _NPAIR
_NROW_PAD = 656
_NTILE = 32
_BPT = _B // _NTILE
_CH = 32
_NCH = _BPT // _CH


def _ffm_body(table, ipairs, idxp, wlin, blin, out,
              ip_v, ix_v, rows_v, wlin_v, blin_v, out_v, sem):
    nc = 2
    wid = lax.axis_index("s") * nc + lax.axis_index("c")
    base = wid * _BPT

    pltpu.sync_copy(wlin, wlin_v)
    pltpu.sync_copy(blin, blin_v)
    b0vec = blin_v[pl.ds(0, 16)]
    lane = lax.iota(jnp.int32, 16)

    for ch in range(_NCH):
        b0 = base + ch * _CH
        pltpu.sync_copy(ipairs.at[pl.ds(b0, _CH)], ip_v)
        pltpu.sync_copy(idxp.at[pl.ds(b0, _CH)], ix_v)

        def body(lb, res):
            copies = []
            for c in range(6):
                off = c * 128
                n = 128 if c < 5 else _NROW_PAD - 5 * 128
                copies.append(pltpu.async_copy(
                    table.at[ip_v.at[lb, pl.ds(off, n)]],
                    rows_v.at[pl.ds(off, n)], sem))
            for cp in copies:
                cp.wait()

            def pbody(i, acc):
                for u in range(5):
                    p = i * 5 + u
                    l1 = rows_v[2 * p, pl.ds(0, 16)]
                    r1 = rows_v[2 * p + 1, pl.ds(0, 16)]
                    l2 = rows_v[2 * p, pl.ds(16, 16)]
                    r2 = rows_v[2 * p + 1, pl.ds(16, 16)]
                    acc = acc + l1 * r1 + l2 * r2
                return acc

            acc = lax.fori_loop(0, _NPAIR // 5, pbody,
                                jnp.zeros((16,), jnp.float32))
            g1 = plsc.load_gather(wlin_v, [ix_v[lb, pl.ds(0, 16)]])
            g2 = plsc.load_gather(wlin_v, [ix_v[lb, pl.ds(16, 16)]])
            tot = acc + g1 + g2
            s = jnp.sum(tot) + b0vec[0]
            res = jnp.where(lane == (lb & 15), s, res)

            @pl.when((lb & 15) == 15)
            def _():
                out_v[pl.ds(ch * _CH + lb - 15, 16)] = res

            return res

        lax.fori_loop(0, _CH, body, jnp.zeros((16,), jnp.float32))

    pltpu.sync_copy(out_v, out.at[pl.ds(base, _BPT)])


@jax.jit
def kernel(x, W_lin, b_lin, W_emb):
    offs = jnp.asarray(_OFFS)
    idx = x + offs[None, :]
    li = idx[:, _I] + jnp.asarray(_J * _V, dtype=jnp.int32)
    ri = idx[:, _J] + jnp.asarray(_I * _V, dtype=jnp.int32)
    inter = jnp.stack([li, ri], axis=2).reshape(_B, _NROW)
    ipairs = jnp.concatenate(
        [inter, jnp.zeros((_B, _NROW_PAD - _NROW), jnp.int32)], axis=1)
    idxp = jnp.concatenate(
        [idx, jnp.full((_B, 32 - _F), _V, jnp.int32)], axis=1)
    table = W_emb.reshape(_F * _V, _D)
    wlin_pad = jnp.concatenate([W_lin[:, 0], jnp.zeros((8,), jnp.float32)])
    blin_pad = jnp.concatenate([b_lin, jnp.zeros((15,), jnp.float32)])

    mesh = plsc.VectorSubcoreMesh(core_axis_name="c", subcore_axis_name="s")
    run = functools.partial(
        pl.kernel, _ffm_body,
        out_type=jax.ShapeDtypeStruct((_B,), jnp.float32),
        mesh=mesh,
        compiler_params=pltpu.CompilerParams(
            needs_layout_passes=False, use_tc_tiling_on_sc=False),
        scratch_types=[
            pltpu.VMEM((_CH, _NROW_PAD), jnp.int32),
            pltpu.VMEM((_CH, 32), jnp.int32),
            pltpu.VMEM((_NROW_PAD, _D), jnp.float32),
            pltpu.VMEM((_V + 8,), jnp.float32),
            pltpu.VMEM((16,), jnp.float32),
            pltpu.VMEM((_BPT,), jnp.float32),
            pltpu.SemaphoreType.DMA,
        ],
    )()
    return run(table, ipairs, idxp, wlin_pad, blin_pad)
